# baseline (speedup 1.0000x reference)
import functools

import jax
import jax.numpy as jnp
from jax import lax
from jax.experimental import pallas as pl
from jax.experimental.pallas import tpu as pltpu
from jax.experimental.pallas import tpu_sc as plsc

_B, _H2, _W2, _C, _NCLS = 4, 112, 112, 384, 19
_H, _W = 2 * _H2, 2 * _W2
_ROWS = _B * _H * _W
_NW = 32
_CHUNK = 64
_CPW = _ROWS // (_NW * _CHUNK)


def _encode(seg):
    w = (lax.broadcasted_iota(jnp.int32, (_NCLS, 1), 0) + 1).astype(jnp.float32)
    m = jnp.max(seg, axis=1)
    s = jnp.sum(jnp.where(seg == m[:, None, :], w, 0.0), axis=1)
    return m * s


def _encode_body(sd_ref, su_ref, sdc_ref, suc_ref):
    sdc_ref[0] = _encode(sd_ref[0])
    suc_ref[0] = _encode(su_ref[0])


_RD = 16
_RU = 2 * _RD


def _encode_codes(seg_d, seg_u, interpret=False):
    sd_t = seg_d.transpose(0, 1, 3, 2) + 0.0
    su_t = seg_u.transpose(0, 1, 3, 2) + 0.0
    return pl.pallas_call(
        _encode_body,
        grid=(_B, _H2 // _RD),
        in_specs=[
            pl.BlockSpec((1, _RD, _NCLS, _W2), lambda b, r: (b, r, 0, 0)),
            pl.BlockSpec((1, _RU, _NCLS, _W), lambda b, r: (b, r, 0, 0)),
        ],
        out_specs=[
            pl.BlockSpec((1, _RD, _W2), lambda b, r: (b, r, 0)),
            pl.BlockSpec((1, _RU, _W), lambda b, r: (b, r, 0)),
        ],
        out_shape=[
            jax.ShapeDtypeStruct((_B, _H2, _W2), jnp.float32),
            jax.ShapeDtypeStruct((_B, _H, _W), jnp.float32),
        ],
        interpret=interpret,
    )(sd_t, su_t)


def _idx_body(sdc_ref, suc_ref, idx_ref):
    b = pl.program_id(0)

    sd = sdc_ref[0]
    su = suc_ref[0]

    hh = lax.broadcasted_iota(jnp.int32, (_H, _H2), 0)
    cc = lax.broadcasted_iota(jnp.int32, (_H, _H2), 1)
    v0 = (cc == hh // 2).astype(jnp.float32)
    v1 = (cc == hh // 2 + 1).astype(jnp.float32)
    rr = lax.broadcasted_iota(jnp.int32, (_W2, _W), 0)
    ww = lax.broadcasted_iota(jnp.int32, (_W2, _W), 1)
    u0 = (rr == ww // 2).astype(jnp.float32)
    u1 = (rr == ww // 2 + 1).astype(jnp.float32)

    dot = functools.partial(jnp.dot, precision=lax.Precision.HIGHEST,
                            preferred_element_type=jnp.float32)
    t0 = dot(v0, sd)
    t1 = dot(v1, sd)
    c00 = dot(t0, u0)
    c01 = dot(t0, u1)
    c10 = dot(t1, u0)
    c11 = dot(t1, u1)

    ii = lax.broadcasted_iota(jnp.int32, (_H, _W), 0) >> 1
    jj = lax.broadcasted_iota(jnp.int32, (_H, _W), 1) >> 1
    base = b * (_H2 * _W2)
    cand0 = base + ii * _W2 + jj
    cand1 = jnp.where(jj == _W2 - 1, base, cand0 + 1)
    cand2 = jnp.where(ii == _H2 - 1, base, cand0 + _W2)
    cand3 = jnp.where((ii == _H2 - 1) | (jj == _W2 - 1), base, cand0 + _W2 + 1)

    idx = jnp.where(su == c00, cand0,
          jnp.where(su == c01, cand1,
          jnp.where(su == c10, cand2,
          jnp.where(su == c11, cand3, cand0))))
    idx_ref[0] = idx


def _compute_indices(seg_d, seg_u, interpret=False):
    sdc, suc = _encode_codes(seg_d, seg_u, interpret=interpret)
    return pl.pallas_call(
        _idx_body,
        grid=(_B,),
        in_specs=[
            pl.BlockSpec((1, _H2, _W2), lambda b: (b, 0, 0)),
            pl.BlockSpec((1, _H, _W), lambda b: (b, 0, 0)),
        ],
        out_specs=pl.BlockSpec((1, _H, _W), lambda b: (b, 0, 0)),
        out_shape=jax.ShapeDtypeStruct((_B, _H, _W), jnp.int32),
        interpret=interpret,
    )(sdc, suc)


def _gather_body(x_hbm, idx_hbm, out_hbm, idx_v, b0, b1, b2, b3,
                 g0, g1, g2, g3, s0, s1, s2, s3):
    bufs = (b0, b1, b2, b3)
    gsem = (g0, g1, g2, g3)
    ssem = (s0, s1, s2, s3)
    wid = lax.axis_index("s") * 2 + lax.axis_index("c")
    wrow = wid * (_CPW * _CHUNK)

    pltpu.sync_copy(idx_hbm.at[wid], idx_v)

    def start_gather(c, k):
        pltpu.async_copy(x_hbm.at[idx_v.at[c]], bufs[k], gsem[k])

    def wait_gather(k):
        pltpu.make_async_copy(x_hbm.at[idx_v.at[0]], bufs[k], gsem[k]).wait()

    def start_scatter(c, k):
        pltpu.async_copy(bufs[k], out_hbm.at[pl.ds(wrow + c * _CHUNK, _CHUNK)],
                         ssem[k])

    def wait_scatter(k):
        pltpu.make_async_copy(bufs[k], out_hbm.at[pl.ds(wrow, _CHUNK)],
                              ssem[k]).wait()

    start_gather(0, 0)
    start_gather(1, 1)
    start_gather(2, 2)

    wait_gather(0); start_scatter(0, 0); start_gather(3, 3)
    wait_gather(1); start_scatter(1, 1); wait_scatter(0); start_gather(4, 0)
    wait_gather(2); start_scatter(2, 2); wait_scatter(1); start_gather(5, 1)
    wait_gather(3); start_scatter(3, 3); wait_scatter(2); start_gather(6, 2)

    def group(t, carry):
        for k in range(4):
            c = 4 * t + k
            wait_gather(k)
            start_scatter(c, k)
            kp = (k + 3) % 4
            wait_scatter(kp)
            start_gather(c + 3, kp)
        return carry

    lax.fori_loop(1, (_CPW - 6) // 4, group, 0, unroll=False)

    n = _CPW - 6
    wait_gather(n % 4); start_scatter(n, n % 4)
    wait_scatter((n + 3) % 4); start_gather(n + 3, (n + 3) % 4)
    n += 1
    wait_gather(n % 4); start_scatter(n, n % 4)
    wait_scatter((n + 3) % 4); start_gather(n + 3, (n + 3) % 4)
    n += 1
    wait_gather(n % 4); start_scatter(n, n % 4)
    wait_scatter((n + 3) % 4); start_gather(n + 3, (n + 3) % 4)
    for c in range(_CPW - 3, _CPW):
        wait_gather(c % 4)
        start_scatter(c, c % 4)
    for c in range(_CPW - 4, _CPW):
        wait_scatter(c % 4)


@functools.cache
def _sc_gather():
    return pl.kernel(
        _gather_body,
        out_type=jax.ShapeDtypeStruct((_ROWS, _C), jnp.float32),
        mesh=plsc.VectorSubcoreMesh(core_axis_name="c", subcore_axis_name="s"),
        scratch_types=[
            pltpu.VMEM((_CPW, _CHUNK), jnp.int32),
            pltpu.VMEM((_CHUNK, _C), jnp.float32),
            pltpu.VMEM((_CHUNK, _C), jnp.float32),
            pltpu.VMEM((_CHUNK, _C), jnp.float32),
            pltpu.VMEM((_CHUNK, _C), jnp.float32),
            pltpu.SemaphoreType.DMA,
            pltpu.SemaphoreType.DMA,
            pltpu.SemaphoreType.DMA,
            pltpu.SemaphoreType.DMA,
            pltpu.SemaphoreType.DMA,
            pltpu.SemaphoreType.DMA,
            pltpu.SemaphoreType.DMA,
            pltpu.SemaphoreType.DMA,
        ],
    )


@jax.jit
def kernel(x, seg_d, seg_u):
    idx = _compute_indices(seg_d, seg_u)
    idx = idx.reshape(_NW, _CPW, _CHUNK)
    out = _sc_gather()(x.reshape(_B * _H2 * _W2, _C), idx)
    return out.reshape(_B, _H, _W, _C)


# speedup vs baseline: 1.2237x; 1.2237x over previous
import functools

import jax
import jax.numpy as jnp
from jax import lax
from jax.experimental import pallas as pl
from jax.experimental.pallas import tpu as pltpu
from jax.experimental.pallas import tpu_sc as plsc

_B, _H2, _W2, _C, _NCLS = 4, 112, 112, 384, 19
_H, _W = 2 * _H2, 2 * _W2
_ROWS = _B * _H * _W
_NW = 32
_CHUNK = 64
_CPW = _ROWS // (_NW * _CHUNK)


def _encode(seg):
    w = (lax.broadcasted_iota(jnp.int32, (_NCLS, 1), 0) + 1).astype(jnp.float32)
    m = jnp.max(seg, axis=1)
    s = jnp.sum(jnp.where(seg == m[:, None, :], w, 0.0), axis=1)
    return m * s


def _encode_body(sd_ref, su_ref, sdc_ref, suc_ref):
    sdc_ref[0] = _encode(sd_ref[0])
    suc_ref[0] = _encode(su_ref[0])


_RD = 16
_RU = 2 * _RD


def _encode_codes(seg_d, seg_u, interpret=False):
    sd_t = jnp.maximum(seg_d.transpose(0, 1, 3, 2), 0.0)
    su_t = jnp.maximum(seg_u.transpose(0, 1, 3, 2), 0.0)
    return pl.pallas_call(
        _encode_body,
        grid=(_B, _H2 // _RD),
        in_specs=[
            pl.BlockSpec((1, _RD, _NCLS, _W2), lambda b, r: (b, r, 0, 0)),
            pl.BlockSpec((1, _RU, _NCLS, _W), lambda b, r: (b, r, 0, 0)),
        ],
        out_specs=[
            pl.BlockSpec((1, _RD, _W2), lambda b, r: (b, r, 0)),
            pl.BlockSpec((1, _RU, _W), lambda b, r: (b, r, 0)),
        ],
        out_shape=[
            jax.ShapeDtypeStruct((_B, _H2, _W2), jnp.float32),
            jax.ShapeDtypeStruct((_B, _H, _W), jnp.float32),
        ],
        interpret=interpret,
    )(sd_t, su_t)


def _idx_body(sdc_ref, suc_ref, idx_ref):
    b = pl.program_id(0)

    sd = sdc_ref[0]
    su = suc_ref[0]

    hh = lax.broadcasted_iota(jnp.int32, (_H, _H2), 0)
    cc = lax.broadcasted_iota(jnp.int32, (_H, _H2), 1)
    v0 = (cc == hh // 2).astype(jnp.float32)
    v1 = (cc == hh // 2 + 1).astype(jnp.float32)
    rr = lax.broadcasted_iota(jnp.int32, (_W2, _W), 0)
    ww = lax.broadcasted_iota(jnp.int32, (_W2, _W), 1)
    u0 = (rr == ww // 2).astype(jnp.float32)
    u1 = (rr == ww // 2 + 1).astype(jnp.float32)

    dot = functools.partial(jnp.dot, precision=lax.Precision.HIGHEST,
                            preferred_element_type=jnp.float32)
    t0 = dot(v0, sd)
    t1 = dot(v1, sd)
    c00 = dot(t0, u0)
    c01 = dot(t0, u1)
    c10 = dot(t1, u0)
    c11 = dot(t1, u1)

    ii = lax.broadcasted_iota(jnp.int32, (_H, _W), 0) >> 1
    jj = lax.broadcasted_iota(jnp.int32, (_H, _W), 1) >> 1
    base = b * (_H2 * _W2)
    cand0 = base + ii * _W2 + jj
    cand1 = jnp.where(jj == _W2 - 1, base, cand0 + 1)
    cand2 = jnp.where(ii == _H2 - 1, base, cand0 + _W2)
    cand3 = jnp.where((ii == _H2 - 1) | (jj == _W2 - 1), base, cand0 + _W2 + 1)

    idx = jnp.where(su == c00, cand0,
          jnp.where(su == c01, cand1,
          jnp.where(su == c10, cand2,
          jnp.where(su == c11, cand3, cand0))))
    idx_ref[0] = idx


def _compute_indices(seg_d, seg_u, interpret=False):
    sdc, suc = _encode_codes(seg_d, seg_u, interpret=interpret)
    return pl.pallas_call(
        _idx_body,
        grid=(_B,),
        in_specs=[
            pl.BlockSpec((1, _H2, _W2), lambda b: (b, 0, 0)),
            pl.BlockSpec((1, _H, _W), lambda b: (b, 0, 0)),
        ],
        out_specs=pl.BlockSpec((1, _H, _W), lambda b: (b, 0, 0)),
        out_shape=jax.ShapeDtypeStruct((_B, _H, _W), jnp.int32),
        interpret=interpret,
    )(sdc, suc)


def _gather_body(x_hbm, idx_hbm, out_hbm, idx_v, b0, b1, b2, b3,
                 g0, g1, g2, g3, s0, s1, s2, s3):
    bufs = (b0, b1, b2, b3)
    gsem = (g0, g1, g2, g3)
    ssem = (s0, s1, s2, s3)
    wid = lax.axis_index("s") * 2 + lax.axis_index("c")
    wrow = wid * (_CPW * _CHUNK)

    pltpu.sync_copy(idx_hbm.at[wid], idx_v)

    def start_gather(c, k):
        pltpu.async_copy(x_hbm.at[idx_v.at[c]], bufs[k], gsem[k])

    def wait_gather(k):
        pltpu.make_async_copy(x_hbm.at[idx_v.at[0]], bufs[k], gsem[k]).wait()

    def start_scatter(c, k):
        pltpu.async_copy(bufs[k], out_hbm.at[pl.ds(wrow + c * _CHUNK, _CHUNK)],
                         ssem[k])

    def wait_scatter(k):
        pltpu.make_async_copy(bufs[k], out_hbm.at[pl.ds(wrow, _CHUNK)],
                              ssem[k]).wait()

    start_gather(0, 0)
    start_gather(1, 1)
    start_gather(2, 2)

    wait_gather(0); start_scatter(0, 0); start_gather(3, 3)
    wait_gather(1); start_scatter(1, 1); wait_scatter(0); start_gather(4, 0)
    wait_gather(2); start_scatter(2, 2); wait_scatter(1); start_gather(5, 1)
    wait_gather(3); start_scatter(3, 3); wait_scatter(2); start_gather(6, 2)

    def group(t, carry):
        for k in range(4):
            c = 4 * t + k
            wait_gather(k)
            start_scatter(c, k)
            kp = (k + 3) % 4
            wait_scatter(kp)
            start_gather(c + 3, kp)
        return carry

    lax.fori_loop(1, (_CPW - 6) // 4, group, 0, unroll=False)

    n = _CPW - 6
    wait_gather(n % 4); start_scatter(n, n % 4)
    wait_scatter((n + 3) % 4); start_gather(n + 3, (n + 3) % 4)
    n += 1
    wait_gather(n % 4); start_scatter(n, n % 4)
    wait_scatter((n + 3) % 4); start_gather(n + 3, (n + 3) % 4)
    n += 1
    wait_gather(n % 4); start_scatter(n, n % 4)
    wait_scatter((n + 3) % 4); start_gather(n + 3, (n + 3) % 4)
    for c in range(_CPW - 3, _CPW):
        wait_gather(c % 4)
        start_scatter(c, c % 4)
    for c in range(_CPW - 4, _CPW):
        wait_scatter(c % 4)


@functools.cache
def _sc_gather():
    return pl.kernel(
        _gather_body,
        out_type=jax.ShapeDtypeStruct((_ROWS, _C), jnp.float32),
        mesh=plsc.VectorSubcoreMesh(core_axis_name="c", subcore_axis_name="s"),
        scratch_types=[
            pltpu.VMEM((_CPW, _CHUNK), jnp.int32),
            pltpu.VMEM((_CHUNK, _C), jnp.float32),
            pltpu.VMEM((_CHUNK, _C), jnp.float32),
            pltpu.VMEM((_CHUNK, _C), jnp.float32),
            pltpu.VMEM((_CHUNK, _C), jnp.float32),
            pltpu.SemaphoreType.DMA,
            pltpu.SemaphoreType.DMA,
            pltpu.SemaphoreType.DMA,
            pltpu.SemaphoreType.DMA,
            pltpu.SemaphoreType.DMA,
            pltpu.SemaphoreType.DMA,
            pltpu.SemaphoreType.DMA,
            pltpu.SemaphoreType.DMA,
        ],
    )


@jax.jit
def kernel(x, seg_d, seg_u):
    idx = _compute_indices(seg_d, seg_u)
    idx = idx.reshape(_NW, _CPW, _CHUNK)
    out = _sc_gather()(x.reshape(_B * _H2 * _W2, _C), idx)
    return out.reshape(_B, _H, _W, _C)


# speedup vs baseline: 1.2644x; 1.0333x over previous
import functools

import jax
import jax.numpy as jnp
from jax import lax
from jax.experimental import pallas as pl
from jax.experimental.pallas import tpu as pltpu
from jax.experimental.pallas import tpu_sc as plsc

_B, _H2, _W2, _C, _NCLS = 4, 112, 112, 384, 19
_H, _W = 2 * _H2, 2 * _W2
_ROWS = _B * _H * _W
_NW = 32
_CHUNK = 64
_CPW = _ROWS // (_NW * _CHUNK)


def _encode(seg):
    w = (lax.broadcasted_iota(jnp.int32, (_NCLS, 1), 0) + 1).astype(jnp.float32)
    m = jnp.max(seg, axis=1)
    s = jnp.sum(jnp.where(seg == m[:, None, :], w, 0.0), axis=1)
    return m * s


def _idx_body(sd_ref, su_ref, idx_ref):
    b = pl.program_id(0)

    sd = _encode(sd_ref[0])
    su = _encode(su_ref[0])

    hh = lax.broadcasted_iota(jnp.int32, (_H, _H2), 0)
    cc = lax.broadcasted_iota(jnp.int32, (_H, _H2), 1)
    v0 = (cc == hh // 2).astype(jnp.float32)
    v1 = (cc == hh // 2 + 1).astype(jnp.float32)
    rr = lax.broadcasted_iota(jnp.int32, (_W2, _W), 0)
    ww = lax.broadcasted_iota(jnp.int32, (_W2, _W), 1)
    u0 = (rr == ww // 2).astype(jnp.float32)
    u1 = (rr == ww // 2 + 1).astype(jnp.float32)

    dot = functools.partial(jnp.dot, precision=lax.Precision.HIGHEST,
                            preferred_element_type=jnp.float32)
    t0 = dot(v0, sd)
    t1 = dot(v1, sd)
    c00 = dot(t0, u0)
    c01 = dot(t0, u1)
    c10 = dot(t1, u0)
    c11 = dot(t1, u1)

    ii = lax.broadcasted_iota(jnp.int32, (_H, _W), 0) >> 1
    jj = lax.broadcasted_iota(jnp.int32, (_H, _W), 1) >> 1
    base = b * (_H2 * _W2)
    cand0 = base + ii * _W2 + jj
    cand1 = jnp.where(jj == _W2 - 1, base, cand0 + 1)
    cand2 = jnp.where(ii == _H2 - 1, base, cand0 + _W2)
    cand3 = jnp.where((ii == _H2 - 1) | (jj == _W2 - 1), base, cand0 + _W2 + 1)

    idx = jnp.where(su == c00, cand0,
          jnp.where(su == c01, cand1,
          jnp.where(su == c10, cand2,
          jnp.where(su == c11, cand3, cand0))))
    idx_ref[0] = idx


def _compute_indices(seg_d, seg_u, interpret=False):
    sd_t = jnp.maximum(seg_d.transpose(0, 1, 3, 2), 0.0)
    su_t = jnp.maximum(seg_u.transpose(0, 1, 3, 2), 0.0)
    return pl.pallas_call(
        _idx_body,
        grid=(_B,),
        in_specs=[
            pl.BlockSpec((1, _H2, _NCLS, _W2), lambda b: (b, 0, 0, 0)),
            pl.BlockSpec((1, _H, _NCLS, _W), lambda b: (b, 0, 0, 0)),
        ],
        out_specs=pl.BlockSpec((1, _H, _W), lambda b: (b, 0, 0)),
        out_shape=jax.ShapeDtypeStruct((_B, _H, _W), jnp.int32),
        interpret=interpret,
    )(sd_t, su_t)


def _gather_body(x_hbm, idx_hbm, out_hbm, idx_v, b0, b1, b2, b3,
                 g0, g1, g2, g3, s0, s1, s2, s3):
    bufs = (b0, b1, b2, b3)
    gsem = (g0, g1, g2, g3)
    ssem = (s0, s1, s2, s3)
    wid = lax.axis_index("s") * 2 + lax.axis_index("c")
    wrow = wid * (_CPW * _CHUNK)

    pltpu.sync_copy(idx_hbm.at[wid], idx_v)

    def start_gather(c, k):
        pltpu.async_copy(x_hbm.at[idx_v.at[c]], bufs[k], gsem[k])

    def wait_gather(k):
        pltpu.make_async_copy(x_hbm.at[idx_v.at[0]], bufs[k], gsem[k]).wait()

    def start_scatter(c, k):
        pltpu.async_copy(bufs[k], out_hbm.at[pl.ds(wrow + c * _CHUNK, _CHUNK)],
                         ssem[k])

    def wait_scatter(k):
        pltpu.make_async_copy(bufs[k], out_hbm.at[pl.ds(wrow, _CHUNK)],
                              ssem[k]).wait()

    start_gather(0, 0)
    start_gather(1, 1)
    start_gather(2, 2)

    wait_gather(0); start_scatter(0, 0); start_gather(3, 3)
    wait_gather(1); start_scatter(1, 1); wait_scatter(0); start_gather(4, 0)
    wait_gather(2); start_scatter(2, 2); wait_scatter(1); start_gather(5, 1)
    wait_gather(3); start_scatter(3, 3); wait_scatter(2); start_gather(6, 2)

    def group(t, carry):
        for k in range(4):
            c = 4 * t + k
            wait_gather(k)
            start_scatter(c, k)
            kp = (k + 3) % 4
            wait_scatter(kp)
            start_gather(c + 3, kp)
        return carry

    lax.fori_loop(1, (_CPW - 6) // 4, group, 0, unroll=False)

    n = _CPW - 6
    wait_gather(n % 4); start_scatter(n, n % 4)
    wait_scatter((n + 3) % 4); start_gather(n + 3, (n + 3) % 4)
    n += 1
    wait_gather(n % 4); start_scatter(n, n % 4)
    wait_scatter((n + 3) % 4); start_gather(n + 3, (n + 3) % 4)
    n += 1
    wait_gather(n % 4); start_scatter(n, n % 4)
    wait_scatter((n + 3) % 4); start_gather(n + 3, (n + 3) % 4)
    for c in range(_CPW - 3, _CPW):
        wait_gather(c % 4)
        start_scatter(c, c % 4)
    for c in range(_CPW - 4, _CPW):
        wait_scatter(c % 4)


@functools.cache
def _sc_gather():
    return pl.kernel(
        _gather_body,
        out_type=jax.ShapeDtypeStruct((_ROWS, _C), jnp.float32),
        mesh=plsc.VectorSubcoreMesh(core_axis_name="c", subcore_axis_name="s"),
        scratch_types=[
            pltpu.VMEM((_CPW, _CHUNK), jnp.int32),
            pltpu.VMEM((_CHUNK, _C), jnp.float32),
            pltpu.VMEM((_CHUNK, _C), jnp.float32),
            pltpu.VMEM((_CHUNK, _C), jnp.float32),
            pltpu.VMEM((_CHUNK, _C), jnp.float32),
            pltpu.SemaphoreType.DMA,
            pltpu.SemaphoreType.DMA,
            pltpu.SemaphoreType.DMA,
            pltpu.SemaphoreType.DMA,
            pltpu.SemaphoreType.DMA,
            pltpu.SemaphoreType.DMA,
            pltpu.SemaphoreType.DMA,
            pltpu.SemaphoreType.DMA,
        ],
    )


@jax.jit
def kernel(x, seg_d, seg_u):
    idx = _compute_indices(seg_d, seg_u)
    idx = idx.reshape(_NW, _CPW, _CHUNK)
    out = _sc_gather()(x.reshape(_B * _H2 * _W2, _C), idx)
    return out.reshape(_B, _H, _W, _C)
